# baseline (speedup 1.0000x reference)
import functools

import jax
import jax.numpy as jnp
from jax import lax
from jax.experimental import pallas as pl
from jax.experimental.pallas import tpu as pltpu
from jax.experimental.pallas import tpu_sc as plsc

_NUM_CLASSES = 1000
_CTX_LEN = 4
_PROMPT_LEN = _CTX_LEN + 1
_EMB_DIM = 64
_LANES = 16
_VPR = _EMB_DIM // _LANES

_info = plsc.get_sparse_core_info()
_NC = _info.num_cores
_NS = _info.num_subcores
_NW = _NC * _NS
_CPW = 32


def _body(ctx_hbm, emb_hbm, out_hbm, ctx_v, emb_v, block_v):
    wid = lax.axis_index("s") * _NC + lax.axis_index("c")
    base = jnp.minimum(wid * _CPW, _NUM_CLASSES - _CPW)
    pltpu.sync_copy(ctx_hbm, ctx_v)
    pltpu.sync_copy(emb_hbm.at[pl.ds(base, _CPW)], emb_v)
    for j in range(_CTX_LEN):
        for k in range(_VPR):
            reg = ctx_v[j, pl.ds(k * _LANES, _LANES)]
            for c in range(_CPW):
                block_v[c, j, pl.ds(k * _LANES, _LANES)] = reg
    for c in range(_CPW):
        for k in range(_VPR):
            block_v[c, _CTX_LEN, pl.ds(k * _LANES, _LANES)] = emb_v[
                c, pl.ds(k * _LANES, _LANES)
            ]
    pltpu.sync_copy(block_v, out_hbm.at[pl.ds(base, _CPW)])


@functools.partial(
    pl.kernel,
    mesh=plsc.VectorSubcoreMesh(core_axis_name="c", subcore_axis_name="s"),
    out_type=jax.ShapeDtypeStruct((_NUM_CLASSES, _PROMPT_LEN, _EMB_DIM), jnp.float32),
    scratch_types=[
        pltpu.VMEM((_CTX_LEN, _EMB_DIM), jnp.float32),
        pltpu.VMEM((_CPW, _EMB_DIM), jnp.float32),
        pltpu.VMEM((_CPW, _PROMPT_LEN, _EMB_DIM), jnp.float32),
    ],
)
def _sc_prompt_head(ctx_hbm, emb_hbm, out_hbm, ctx_v, emb_v, block_v):
    _body(ctx_hbm, emb_hbm, out_hbm, ctx_v, emb_v, block_v)


@jax.jit
def kernel(context, emb_table):
    return _sc_prompt_head(context, emb_table)


# speedup vs baseline: 1.0069x; 1.0069x over previous
import functools

import jax
import jax.numpy as jnp
from jax import lax
from jax.experimental import pallas as pl
from jax.experimental.pallas import tpu as pltpu
from jax.experimental.pallas import tpu_sc as plsc

_NUM_CLASSES = 1000
_CTX_LEN = 4
_PROMPT_LEN = _CTX_LEN + 1
_EMB_DIM = 64
_LANES = 16
_VPR = _EMB_DIM // _LANES

_info = plsc.get_sparse_core_info()
_NC = _info.num_cores
_NS = _info.num_subcores
_NW = _NC * _NS
_CPW = 32


def _body(ctx_hbm, emb_hbm, out_hbm, ctx_v, block_v, sem_ctx, sem_emb):
    wid = lax.axis_index("s") * _NC + lax.axis_index("c")
    base = jnp.minimum(wid * _CPW, _NUM_CLASSES - _CPW)
    cp_ctx = pltpu.async_copy(ctx_hbm, ctx_v, sem_ctx)
    cp_emb = pltpu.async_copy(
        emb_hbm.at[pl.ds(base, _CPW)], block_v.at[:, _CTX_LEN, :], sem_emb
    )
    cp_ctx.wait()
    for j in range(_CTX_LEN):
        for k in range(_VPR):
            reg = ctx_v[j, pl.ds(k * _LANES, _LANES)]
            for c in range(_CPW):
                block_v[c, j, pl.ds(k * _LANES, _LANES)] = reg
    cp_emb.wait()
    pltpu.sync_copy(block_v, out_hbm.at[pl.ds(base, _CPW)])


@functools.partial(
    pl.kernel,
    mesh=plsc.VectorSubcoreMesh(core_axis_name="c", subcore_axis_name="s"),
    out_type=jax.ShapeDtypeStruct((_NUM_CLASSES, _PROMPT_LEN, _EMB_DIM), jnp.float32),
    scratch_types=[
        pltpu.VMEM((_CTX_LEN, _EMB_DIM), jnp.float32),
        pltpu.VMEM((_CPW, _PROMPT_LEN, _EMB_DIM), jnp.float32),
        pltpu.SemaphoreType.DMA,
        pltpu.SemaphoreType.DMA,
    ],
)
def _sc_prompt_head(ctx_hbm, emb_hbm, out_hbm, ctx_v, block_v, sem_ctx, sem_emb):
    _body(ctx_hbm, emb_hbm, out_hbm, ctx_v, block_v, sem_ctx, sem_emb)


@jax.jit
def kernel(context, emb_table):
    return _sc_prompt_head(context, emb_table)


# speedup vs baseline: 1.0213x; 1.0143x over previous
import functools

import jax
import jax.numpy as jnp
from jax import lax
from jax.experimental import pallas as pl
from jax.experimental.pallas import tpu as pltpu
from jax.experimental.pallas import tpu_sc as plsc

_NUM_CLASSES = 1000
_CTX_LEN = 4
_PROMPT_LEN = _CTX_LEN + 1
_EMB_DIM = 64
_LANES = 16
_VPR = _EMB_DIM // _LANES

_NC = 1
_NS = 16
_NW = _NC * _NS
_CPW = 64


def _body(ctx_hbm, emb_hbm, out_hbm, ctx_v, block_v, sem_ctx, sem_emb):
    wid = lax.axis_index("s") * _NC + lax.axis_index("c")
    base = jnp.minimum(wid * _CPW, _NUM_CLASSES - _CPW)
    cp_ctx = pltpu.async_copy(ctx_hbm, ctx_v, sem_ctx)
    cp_emb = pltpu.async_copy(
        emb_hbm.at[pl.ds(base, _CPW)], block_v.at[:, _CTX_LEN, :], sem_emb
    )
    cp_ctx.wait()
    for j in range(_CTX_LEN):
        for k in range(_VPR):
            reg = ctx_v[j, pl.ds(k * _LANES, _LANES)]
            for c in range(_CPW):
                block_v[c, j, pl.ds(k * _LANES, _LANES)] = reg
    cp_emb.wait()
    pltpu.sync_copy(block_v, out_hbm.at[pl.ds(base, _CPW)])


@functools.partial(
    pl.kernel,
    mesh=plsc.VectorSubcoreMesh(core_axis_name="c", subcore_axis_name="s", num_cores=_NC),
    out_type=jax.ShapeDtypeStruct((_NUM_CLASSES, _PROMPT_LEN, _EMB_DIM), jnp.float32),
    scratch_types=[
        pltpu.VMEM((_CTX_LEN, _EMB_DIM), jnp.float32),
        pltpu.VMEM((_CPW, _PROMPT_LEN, _EMB_DIM), jnp.float32),
        pltpu.SemaphoreType.DMA,
        pltpu.SemaphoreType.DMA,
    ],
)
def _sc_prompt_head(ctx_hbm, emb_hbm, out_hbm, ctx_v, block_v, sem_ctx, sem_emb):
    _body(ctx_hbm, emb_hbm, out_hbm, ctx_v, block_v, sem_ctx, sem_emb)


@jax.jit
def kernel(context, emb_table):
    return _sc_prompt_head(context, emb_table)


# speedup vs baseline: 1.0269x; 1.0055x over previous
import functools

import jax
import jax.numpy as jnp
from jax import lax
from jax.experimental import pallas as pl
from jax.experimental.pallas import tpu as pltpu
from jax.experimental.pallas import tpu_sc as plsc

_NUM_CLASSES = 1000
_CTX_LEN = 4
_PROMPT_LEN = _CTX_LEN + 1
_EMB_DIM = 64
_LANES = 16
_VPR = _EMB_DIM // _LANES

_NC = 1
_NS = 16
_NW = _NC * _NS
_CPW = 64


_HALF = _CPW // 2


def _body(ctx_hbm, emb_hbm, out_hbm, ctx_v, block_v, sem_ctx, sem_emb, sem_out):
    wid = lax.axis_index("s") * _NC + lax.axis_index("c")
    base = jnp.minimum(wid * _CPW, _NUM_CLASSES - _CPW)
    cp_ctx = pltpu.async_copy(ctx_hbm, ctx_v, sem_ctx)
    cp_emb_a = pltpu.async_copy(
        emb_hbm.at[pl.ds(base, _HALF)],
        block_v.at[pl.ds(0, _HALF), _CTX_LEN, :],
        sem_emb,
    )
    cp_emb_b = pltpu.async_copy(
        emb_hbm.at[pl.ds(base + _HALF, _HALF)],
        block_v.at[pl.ds(_HALF, _HALF), _CTX_LEN, :],
        sem_emb,
    )
    cp_ctx.wait()
    regs = [
        ctx_v[j, pl.ds(k * _LANES, _LANES)]
        for j in range(_CTX_LEN)
        for k in range(_VPR)
    ]
    for c in range(_HALF):
        for j in range(_CTX_LEN):
            for k in range(_VPR):
                block_v[c, j, pl.ds(k * _LANES, _LANES)] = regs[j * _VPR + k]
    cp_emb_a.wait()
    cp_out_a = pltpu.async_copy(
        block_v.at[pl.ds(0, _HALF)], out_hbm.at[pl.ds(base, _HALF)], sem_out
    )
    for c in range(_HALF, _CPW):
        for j in range(_CTX_LEN):
            for k in range(_VPR):
                block_v[c, j, pl.ds(k * _LANES, _LANES)] = regs[j * _VPR + k]
    cp_emb_b.wait()
    cp_out_b = pltpu.async_copy(
        block_v.at[pl.ds(_HALF, _HALF)],
        out_hbm.at[pl.ds(base + _HALF, _HALF)],
        sem_out,
    )
    cp_out_a.wait()
    cp_out_b.wait()


@functools.partial(
    pl.kernel,
    mesh=plsc.VectorSubcoreMesh(core_axis_name="c", subcore_axis_name="s", num_cores=_NC),
    out_type=jax.ShapeDtypeStruct((_NUM_CLASSES, _PROMPT_LEN, _EMB_DIM), jnp.float32),
    scratch_types=[
        pltpu.VMEM((_CTX_LEN, _EMB_DIM), jnp.float32),
        pltpu.VMEM((_CPW, _PROMPT_LEN, _EMB_DIM), jnp.float32),
        pltpu.SemaphoreType.DMA,
        pltpu.SemaphoreType.DMA,
        pltpu.SemaphoreType.DMA,
    ],
)
def _sc_prompt_head(ctx_hbm, emb_hbm, out_hbm, ctx_v, block_v, sem_ctx, sem_emb, sem_out):
    _body(ctx_hbm, emb_hbm, out_hbm, ctx_v, block_v, sem_ctx, sem_emb, sem_out)


@jax.jit
def kernel(context, emb_table):
    return _sc_prompt_head(context, emb_table)
